# Initial kernel scaffold; baseline (speedup 1.0000x reference)
import jax
import jax.numpy as jnp
from jax.experimental import pallas as pl


def kernel(rpn_objectness, rpn_boxoffset, anchors):
    raise NotImplementedError("write your pallas kernel here")


# speedup vs baseline: 1.7981x; 1.7981x over previous
import jax
import jax.numpy as jnp
from jax.experimental import pallas as pl
from jax.experimental.pallas import tpu as pltpu

_N = 2048
_CH = 128
_NCH = _N // _CH
_TOPN = 2000
_IOU = 0.7
_MINSZ = 0.01


def _decode_kernel(sc_ref, tx_ref, ty_ref, tw_ref, th_ref,
                   aw_ref, ah_ref, cx_ref,
                   x1_ref, y1_ref, x2_ref, y2_ref, s_ref):
    rows = jax.lax.broadcasted_iota(jnp.int32, sc_ref.shape, 0)
    cy = ((rows % 64).astype(jnp.float32) + 0.5) * 16.0
    aw = aw_ref[...]
    ah = ah_ref[...]
    cx = cx_ref[...]
    tx = tx_ref[...]
    ty = ty_ref[...]
    pw = jnp.exp(tw_ref[...]) * aw / 1024.0
    ph = jnp.exp(th_ref[...]) * ah / 1024.0
    xc = (aw * tx + cx) / 1024.0
    yc = (ah * ty + cy) / 1024.0
    x1 = jnp.clip(xc - pw / 2.0, 0.0, 1.0)
    y1 = jnp.clip(yc - ph / 2.0, 0.0, 1.0)
    x2 = jnp.clip(xc + pw / 2.0, 0.0, 1.0)
    y2 = jnp.clip(yc + ph / 2.0, 0.0, 1.0)
    v = ((x2 - x1 > _MINSZ) & (y2 - y1 > _MINSZ)).astype(jnp.float32)
    x1_ref[...] = x1 * v
    y1_ref[...] = y1 * v
    x2_ref[...] = x2 * v
    y2_ref[...] = y2 * v
    s_ref[...] = jnp.clip(sc_ref[...], 0.0, 1.0) * v


def _nms_kernel(bT_ref, bs_ref, out_ref, alive_ref, adjl_ref, alc_ref):
    bT = bT_ref[0]
    bs = bs_ref[0]
    x1 = bT[0:1, :]
    y1 = bT[1:2, :]
    x2 = bT[2:3, :]
    y2 = bT[3:4, :]
    areas = (x2 - x1) * (y2 - y1)
    col = jax.lax.broadcasted_iota(jnp.int32, (1, _N), 1)
    lcol = jax.lax.broadcasted_iota(jnp.int32, (1, _CH), 1)
    alive_ref[...] = jnp.ones((1, _N), jnp.float32)

    def chunk_body(c, carry):
        start = c * _CH
        cb = bs_ref[0, pl.ds(start, _CH), :]
        cx1 = cb[:, 0:1]
        cy1 = cb[:, 1:2]
        cx2 = cb[:, 2:3]
        cy2 = cb[:, 3:4]
        carea = (cx2 - cx1) * (cy2 - cy1)
        ix = jnp.minimum(cx2, x2) - jnp.maximum(cx1, x1)
        iy = jnp.minimum(cy2, y2) - jnp.maximum(cy1, y1)
        inter = jnp.maximum(ix, 0.0) * jnp.maximum(iy, 0.0)
        iou = inter / (carea + areas - inter + 1e-9)
        adj = (iou > _IOU).astype(jnp.float32)
        cbT = bT_ref[0, :, pl.ds(start, _CH)]
        rx1 = cbT[0:1, :]
        ry1 = cbT[1:2, :]
        rx2 = cbT[2:3, :]
        ry2 = cbT[3:4, :]
        rarea = (rx2 - rx1) * (ry2 - ry1)
        lix = jnp.minimum(cx2, rx2) - jnp.maximum(cx1, rx1)
        liy = jnp.minimum(cy2, ry2) - jnp.maximum(cy1, ry1)
        linter = jnp.maximum(lix, 0.0) * jnp.maximum(liy, 0.0)
        liou = linter / (carea + rarea - linter + 1e-9)
        adjl_ref[...] = (liou > _IOU).astype(jnp.float32)
        alc_ref[...] = alive_ref[:, pl.ds(start, _CH)]

        def inner(r, carry2):
            arow = adjl_ref[pl.ds(r, 1), :]
            ac = alc_ref[...]
            a_r = jnp.sum(ac * (lcol == r).astype(jnp.float32))
            sup = arow * a_r * (lcol > r).astype(jnp.float32)
            alc_ref[...] = alc_ref[...] * (1.0 - sup)
            return carry2

        jax.lax.fori_loop(0, _CH, inner, 0)
        alive_c = alc_ref[...]
        alive_ref[:, pl.ds(start, _CH)] = alive_c
        counts = jnp.dot(alive_c, adj, preferred_element_type=jnp.float32)
        sup_all = (counts > 0.0) & (col >= start + _CH)
        alive_ref[...] = alive_ref[...] * (1.0 - sup_all.astype(jnp.float32))
        return carry

    jax.lax.fori_loop(0, _NCH, chunk_body, 0)
    alive = alive_ref[...]

    r_i = jax.lax.broadcasted_iota(jnp.int32, (_N, _N), 0)
    c_i = jax.lax.broadcasted_iota(jnp.int32, (_N, _N), 1)
    lt = (r_i <= c_i).astype(jnp.float32)
    c_incl = jnp.dot(alive, lt, preferred_element_type=jnp.float32)
    dest = c_incl - 1.0
    d_i = r_i.astype(jnp.float32)
    P = jnp.where(d_i == dest, 1.0, 0.0) * alive
    out_ref[...] = jnp.dot(P, bs, preferred_element_type=jnp.float32)[None]


@jax.jit
def kernel(rpn_objectness, rpn_boxoffset, anchors):
    b, hs, ws, k = rpn_objectness.shape
    hw = hs * ws
    n_all = hw * k
    sc = rpn_objectness.reshape(b * hs, ws * k)
    off = rpn_boxoffset.reshape(b, hs, ws, k, 4)
    tx = off[..., 0].reshape(b * hs, ws * k)
    ty = off[..., 1].reshape(b * hs, ws * k)
    tw = off[..., 2].reshape(b * hs, ws * k)
    th = off[..., 3].reshape(b * hs, ws * k)
    col = jnp.arange(ws * k)
    aw = anchors[col % k, 0][None, :]
    ah = anchors[col % k, 1][None, :]
    cxv = (((col // k).astype(jnp.float32) + 0.5) * 16.0)[None, :]

    shp = jax.ShapeDtypeStruct((b * hs, ws * k), jnp.float32)
    x1, y1, x2, y2, s = pl.pallas_call(
        _decode_kernel,
        out_shape=(shp, shp, shp, shp, shp),
    )(sc, tx, ty, tw, th, aw, ah, cxv)

    scores = s.reshape(b, n_all)
    _, idx = jax.lax.top_k(scores, _TOPN)
    gx1 = jnp.take_along_axis(x1.reshape(b, n_all), idx, axis=1)
    gy1 = jnp.take_along_axis(y1.reshape(b, n_all), idx, axis=1)
    gx2 = jnp.take_along_axis(x2.reshape(b, n_all), idx, axis=1)
    gy2 = jnp.take_along_axis(y2.reshape(b, n_all), idx, axis=1)

    pad = _N - _TOPN
    bT = jnp.stack([gx1, gy1, gx2, gy2], axis=1)
    bT = jnp.pad(bT, ((0, 0), (0, 0), (0, pad)))
    bs = jnp.stack([gx1, gy1, gx2, gy2], axis=-1)
    bs = jnp.pad(bs, ((0, 0), (0, pad), (0, 0)))

    out = pl.pallas_call(
        _nms_kernel,
        grid=(b,),
        in_specs=[
            pl.BlockSpec((1, 4, _N), lambda i: (i, 0, 0)),
            pl.BlockSpec((1, _N, 4), lambda i: (i, 0, 0)),
        ],
        out_specs=pl.BlockSpec((1, _N, 4), lambda i: (i, 0, 0)),
        out_shape=jax.ShapeDtypeStruct((b, _N, 4), jnp.float32),
        scratch_shapes=[
            pltpu.VMEM((1, _N), jnp.float32),
            pltpu.VMEM((_CH, _CH), jnp.float32),
            pltpu.VMEM((1, _CH), jnp.float32),
        ],
    )(bT, bs)
    return out[:, :_TOPN, :]


# speedup vs baseline: 6.7621x; 3.7606x over previous
import jax
import jax.numpy as jnp
from jax.experimental import pallas as pl
from jax.experimental.pallas import tpu as pltpu

_N = 2048
_CH = 128
_NCH = _N // _CH
_TOPN = 2000
_IOU = 0.7
_MINSZ = 0.01


def _decode_kernel(sc_ref, tx_ref, ty_ref, tw_ref, th_ref,
                   aw_ref, ah_ref, cx_ref,
                   x1_ref, y1_ref, x2_ref, y2_ref, s_ref):
    rows = jax.lax.broadcasted_iota(jnp.int32, sc_ref.shape, 0)
    cy = ((rows % 64).astype(jnp.float32) + 0.5) * 16.0
    aw = aw_ref[...]
    ah = ah_ref[...]
    cx = cx_ref[...]
    tx = tx_ref[...]
    ty = ty_ref[...]
    pw = jnp.exp(tw_ref[...]) * aw / 1024.0
    ph = jnp.exp(th_ref[...]) * ah / 1024.0
    xc = (aw * tx + cx) / 1024.0
    yc = (ah * ty + cy) / 1024.0
    x1 = jnp.clip(xc - pw / 2.0, 0.0, 1.0)
    y1 = jnp.clip(yc - ph / 2.0, 0.0, 1.0)
    x2 = jnp.clip(xc + pw / 2.0, 0.0, 1.0)
    y2 = jnp.clip(yc + ph / 2.0, 0.0, 1.0)
    v = ((x2 - x1 > _MINSZ) & (y2 - y1 > _MINSZ)).astype(jnp.float32)
    x1_ref[...] = x1 * v
    y1_ref[...] = y1 * v
    x2_ref[...] = x2 * v
    y2_ref[...] = y2 * v
    s_ref[...] = jnp.clip(sc_ref[...], 0.0, 1.0) * v


def _nms_kernel(bT_ref, bs_ref, out_ref, alive_ref, adjl_ref, alc_ref, adj_ref):
    nb = bT_ref.shape[0]
    col = jax.lax.broadcasted_iota(jnp.int32, (1, _N), 1)
    lcol = jax.lax.broadcasted_iota(jnp.int32, (1, _CH), 1)
    alive_ref[...] = jnp.ones((nb, _N), jnp.float32)

    def chunk_body(c, carry):
        start = c * _CH
        for img in range(nb):
            bT = bT_ref[img]
            x1 = bT[0:1, :]
            y1 = bT[1:2, :]
            x2 = bT[2:3, :]
            y2 = bT[3:4, :]
            areas = (x2 - x1) * (y2 - y1)
            cb = bs_ref[img, pl.ds(start, _CH), :]
            cx1 = cb[:, 0:1]
            cy1 = cb[:, 1:2]
            cx2 = cb[:, 2:3]
            cy2 = cb[:, 3:4]
            carea = (cx2 - cx1) * (cy2 - cy1)
            ix = jnp.minimum(cx2, x2) - jnp.maximum(cx1, x1)
            iy = jnp.minimum(cy2, y2) - jnp.maximum(cy1, y1)
            inter = jnp.maximum(ix, 0.0) * jnp.maximum(iy, 0.0)
            iou = inter / (carea + areas - inter + 1e-9)
            adj_ref[img] = (iou > _IOU).astype(jnp.float32)
            cbT = bT_ref[img, :, pl.ds(start, _CH)]
            rx1 = cbT[0:1, :]
            ry1 = cbT[1:2, :]
            rx2 = cbT[2:3, :]
            ry2 = cbT[3:4, :]
            rarea = (rx2 - rx1) * (ry2 - ry1)
            lix = jnp.minimum(cx2, rx2) - jnp.maximum(cx1, rx1)
            liy = jnp.minimum(cy2, ry2) - jnp.maximum(cy1, ry1)
            linter = jnp.maximum(lix, 0.0) * jnp.maximum(liy, 0.0)
            liou = linter / (carea + rarea - linter + 1e-9)
            adjl = (liou > _IOU).astype(jnp.float32)
            adjl_ref[:, img : img + 1, :] = adjl[:, None, :]
        alc_ref[...] = alive_ref[:, pl.ds(start, _CH)]

        def inner(r, carry2):
            arow = adjl_ref[pl.ds(r, 1), :, :].reshape(nb, _CH)
            ac = alc_ref[...]
            a_r = jnp.sum(ac * (lcol == r).astype(jnp.float32),
                          axis=1, keepdims=True)
            sup = arow * a_r * (lcol > r).astype(jnp.float32)
            alc_ref[...] = ac * (1.0 - sup)
            return carry2

        jax.lax.fori_loop(0, _CH, inner, 0)
        alive_ref[:, pl.ds(start, _CH)] = alc_ref[...]
        for img in range(nb):
            alive_c = alc_ref[pl.ds(img, 1), :]
            counts = jnp.dot(alive_c, adj_ref[img],
                             preferred_element_type=jnp.float32)
            sup_all = (counts > 0.0) & (col >= start + _CH)
            row = alive_ref[pl.ds(img, 1), :]
            alive_ref[pl.ds(img, 1), :] = row * (1.0 - sup_all.astype(jnp.float32))
        return carry

    jax.lax.fori_loop(0, _NCH, chunk_body, 0)

    r_i = jax.lax.broadcasted_iota(jnp.int32, (_N, _N), 0)
    c_i = jax.lax.broadcasted_iota(jnp.int32, (_N, _N), 1)
    lt = (r_i <= c_i).astype(jnp.float32)
    alive = alive_ref[...]
    c_incl = jnp.dot(alive, lt, preferred_element_type=jnp.float32)
    dest = c_incl - 1.0
    d_i = r_i.astype(jnp.float32)
    for img in range(nb):
        P = jnp.where(d_i == dest[img : img + 1, :], 1.0, 0.0) \
            * alive[img : img + 1, :]
        out_ref[img] = jnp.dot(P, bs_ref[img],
                               preferred_element_type=jnp.float32)


@jax.jit
def kernel(rpn_objectness, rpn_boxoffset, anchors):
    b, hs, ws, k = rpn_objectness.shape
    hw = hs * ws
    n_all = hw * k
    sc = rpn_objectness.reshape(b * hs, ws * k)
    off = rpn_boxoffset.reshape(b, hs, ws, k, 4)
    tx = off[..., 0].reshape(b * hs, ws * k)
    ty = off[..., 1].reshape(b * hs, ws * k)
    tw = off[..., 2].reshape(b * hs, ws * k)
    th = off[..., 3].reshape(b * hs, ws * k)
    col = jnp.arange(ws * k)
    aw = anchors[col % k, 0][None, :]
    ah = anchors[col % k, 1][None, :]
    cxv = (((col // k).astype(jnp.float32) + 0.5) * 16.0)[None, :]

    shp = jax.ShapeDtypeStruct((b * hs, ws * k), jnp.float32)
    x1, y1, x2, y2, s = pl.pallas_call(
        _decode_kernel,
        out_shape=(shp, shp, shp, shp, shp),
    )(sc, tx, ty, tw, th, aw, ah, cxv)

    scores = s.reshape(b, n_all)
    _, idx = jax.lax.top_k(scores, _TOPN)
    gx1 = jnp.take_along_axis(x1.reshape(b, n_all), idx, axis=1)
    gy1 = jnp.take_along_axis(y1.reshape(b, n_all), idx, axis=1)
    gx2 = jnp.take_along_axis(x2.reshape(b, n_all), idx, axis=1)
    gy2 = jnp.take_along_axis(y2.reshape(b, n_all), idx, axis=1)

    pad = _N - _TOPN
    bT = jnp.stack([gx1, gy1, gx2, gy2], axis=1)
    bT = jnp.pad(bT, ((0, 0), (0, 0), (0, pad)))
    bs = jnp.stack([gx1, gy1, gx2, gy2], axis=-1)
    bs = jnp.pad(bs, ((0, 0), (0, pad), (0, 0)))

    out = pl.pallas_call(
        _nms_kernel,
        out_shape=jax.ShapeDtypeStruct((b, _N, 4), jnp.float32),
        scratch_shapes=[
            pltpu.VMEM((b, _N), jnp.float32),
            pltpu.VMEM((_CH, b, _CH), jnp.float32),
            pltpu.VMEM((b, _CH), jnp.float32),
            pltpu.VMEM((b, _CH, _N), jnp.float32),
        ],
    )(bT, bs)
    return out[:, :_TOPN, :]
